# Initial kernel scaffold; baseline (speedup 1.0000x reference)
import jax
import jax.numpy as jnp
from jax.experimental import pallas as pl


def kernel(positions, edge_index, bond_types, batch):
    raise NotImplementedError("write your pallas kernel here")


# speedup vs baseline: 21.8514x; 21.8514x over previous
import jax
import jax.numpy as jnp
import numpy as np
from jax import lax
from jax.experimental import pallas as pl
from jax.experimental.pallas import tpu as pltpu
from jax.experimental.pallas import tpu_sc as plsc

_NC = 2
_NS = 16
_L = 16
_NW = _NC * _NS

_MAGIC = np.int32(0x5F3759DF)


def _sqrt16(s):
    i = lax.bitcast_convert_type(s, jnp.int32)
    i = _MAGIC - lax.shift_right_logical(i, 1)
    y = lax.bitcast_convert_type(i, jnp.float32)
    h = s * jnp.float32(0.5)
    for _ in range(3):
        y = y * (jnp.float32(1.5) - h * y * y)
    return s * y


def _pick_chunk(per_w):
    best = 0
    for c in range(16, 4097, 16):
        if per_w % c == 0:
            best = c
    return best


def _build(n_nodes, e_total, chunk, per_w):
    nchunks = per_w // chunk
    groups = chunk // _L

    def body(pos_hbm, row_hbm, col_hbm, bt_hbm, out_hbm,
             idxr_v, idxc_v, btv_v, posr_v, posc_v, acc_v, sem_i, sem_g):
        wid = lax.axis_index("s") * _NC + lax.axis_index("c")
        base = wid * per_w
        iota = lax.iota(jnp.int32, _L)
        c0 = jnp.zeros((_L,), jnp.int32)
        c1 = jnp.full((_L,), 1, jnp.int32)
        c2 = jnp.full((_L,), 2, jnp.int32)
        acc = jnp.zeros((_L,), jnp.float32)
        for k in range(nchunks):
            off = base + k * chunk
            cp1 = pltpu.async_copy(row_hbm.at[pl.ds(off, chunk)], idxr_v, sem_i)
            cp2 = pltpu.async_copy(col_hbm.at[pl.ds(off, chunk)], idxc_v, sem_i)
            cp3 = pltpu.async_copy(bt_hbm.at[pl.ds(off, chunk)], btv_v, sem_i)
            cp1.wait()
            cp2.wait()
            cp3.wait()
            g1 = pltpu.async_copy(pos_hbm.at[idxr_v], posr_v, sem_g)
            g2 = pltpu.async_copy(pos_hbm.at[idxc_v], posc_v, sem_g)
            g1.wait()
            g2.wait()

            def group(g, acc, off=off):
                eidx = g * _L + iota
                xr = plsc.load_gather(posr_v, [eidx, c0])
                yr = plsc.load_gather(posr_v, [eidx, c1])
                zr = plsc.load_gather(posr_v, [eidx, c2])
                xc = plsc.load_gather(posc_v, [eidx, c0])
                yc = plsc.load_gather(posc_v, [eidx, c1])
                zc = plsc.load_gather(posc_v, [eidx, c2])
                bt = plsc.load_gather(btv_v, [eidx])
                dx = xr - xc
                dy = yr - yc
                dz = zr - zc
                s = dx * dx + dy * dy + dz * dz
                nrm = _sqrt16(s)
                e = jnp.where(bt == 1, jnp.float32(1.34),
                              jnp.where(bt == 2, jnp.float32(1.2),
                                        jnp.where(bt == 3, jnp.float32(1.4),
                                                  jnp.float32(1.54))))
                r = nrm - e
                valid = (off + eidx) < e_total
                r = jnp.where(valid, r, jnp.float32(0.0))
                return acc + r * r

            acc = lax.fori_loop(0, groups, group, acc)
        acc_v[...] = acc
        pltpu.sync_copy(acc_v, out_hbm.at[wid])

    mesh = plsc.VectorSubcoreMesh(core_axis_name="c", subcore_axis_name="s",
                                  num_cores=_NC, num_subcores=_NS)
    return pl.kernel(
        body,
        out_type=jax.ShapeDtypeStruct((_NW, _L), jnp.float32),
        mesh=mesh,
        scratch_types=[
            pltpu.VMEM((chunk,), jnp.int32),
            pltpu.VMEM((chunk,), jnp.int32),
            pltpu.VMEM((chunk,), jnp.int32),
            pltpu.VMEM((chunk, 8), jnp.float32),
            pltpu.VMEM((chunk, 8), jnp.float32),
            pltpu.VMEM((_L,), jnp.float32),
            pltpu.SemaphoreType.DMA,
            pltpu.SemaphoreType.DMA,
        ],
        compiler_params=pltpu.CompilerParams(needs_layout_passes=False,
                                             use_tc_tiling_on_sc=False),
    )


def kernel(positions, edge_index, bond_types, batch):
    del batch
    n_nodes = positions.shape[0]
    e_total = edge_index.shape[1]

    row = edge_index[0].astype(jnp.int32)
    col = edge_index[1].astype(jnp.int32)
    bt = bond_types.astype(jnp.int32)
    pos8 = jnp.concatenate(
        [positions.astype(jnp.float32), jnp.zeros((n_nodes, 5), jnp.float32)],
        axis=1)

    chunk = _pick_chunk(e_total // _NW) if e_total % _NW == 0 else 0
    if chunk:
        per_w = e_total // _NW
    else:
        chunk = 2048
        per_w = -(-e_total // (_NW * chunk)) * chunk
        e_pad = per_w * _NW
        row = jnp.pad(row, (0, e_pad - e_total))
        col = jnp.pad(col, (0, e_pad - e_total))
        bt = jnp.pad(bt, (0, e_pad - e_total))

    run = _build(n_nodes, e_total, chunk, per_w)
    partials = run(pos8, row, col, bt)
    return jnp.sum(partials) / jnp.float32(e_total)


# speedup vs baseline: 26.1511x; 1.1968x over previous
import jax
import jax.numpy as jnp
import numpy as np
from jax import lax
from jax.experimental import pallas as pl
from jax.experimental.pallas import tpu as pltpu
from jax.experimental.pallas import tpu_sc as plsc

_NC = 2
_NS = 16
_L = 16
_NW = _NC * _NS

_MAGIC = np.int32(0x5F3759DF)


def _sqrt16(s):
    i = lax.bitcast_convert_type(s, jnp.int32)
    i = _MAGIC - lax.shift_right_logical(i, 1)
    y = lax.bitcast_convert_type(i, jnp.float32)
    h = s * jnp.float32(0.5)
    for _ in range(3):
        y = y * (jnp.float32(1.5) - h * y * y)
    return s * y


def _pick_chunk(per_w):
    best = 0
    for c in range(16, 4097, 16):
        if per_w % c == 0:
            best = c
    return best


def _build(n_nodes, e_total, chunk, per_w):
    nchunks = per_w // chunk
    groups = chunk // _L
    exact = per_w * _NW == e_total

    def body(pos_hbm, row_hbm, col_hbm, bt_hbm, out_hbm,
             idxr0, idxr1, idxc0, idxc1, btv0, btv1,
             posr0, posr1, posc0, posc1, acc_v, etab_v,
             sem_i0, sem_i1, sem_g0, sem_g1):
        idxr = (idxr0, idxr1)
        idxc = (idxc0, idxc1)
        btv = (btv0, btv1)
        posr = (posr0, posr1)
        posc = (posc0, posc1)
        sem_i = (sem_i0, sem_i1)
        sem_g = (sem_g0, sem_g1)

        wid = lax.axis_index("s") * _NC + lax.axis_index("c")
        base = wid * per_w
        iota = lax.iota(jnp.int32, _L)
        c0 = jnp.zeros((_L,), jnp.int32)
        c1 = jnp.full((_L,), 1, jnp.int32)
        c2 = jnp.full((_L,), 2, jnp.int32)
        etab_v[...] = jnp.where(iota == 1, jnp.float32(1.34),
                                jnp.where(iota == 2, jnp.float32(1.2),
                                          jnp.where(iota == 3, jnp.float32(1.4),
                                                    jnp.float32(1.54))))

        def start_idx(k):
            b = k % 2
            off = base + k * chunk
            return (
                pltpu.async_copy(row_hbm.at[pl.ds(off, chunk)], idxr[b], sem_i[b]),
                pltpu.async_copy(col_hbm.at[pl.ds(off, chunk)], idxc[b], sem_i[b]),
                pltpu.async_copy(bt_hbm.at[pl.ds(off, chunk)], btv[b], sem_i[b]),
            )

        def start_gather(k):
            b = k % 2
            return (
                pltpu.async_copy(pos_hbm.at[idxr[b]], posr[b], sem_g[b]),
                pltpu.async_copy(pos_hbm.at[idxc[b]], posc[b], sem_g[b]),
            )

        def make_group(b, off):
            pr, pc, bv = posr[b], posc[b], btv[b]

            def group(g, acc):
                eidx = g * _L + iota
                xr = plsc.load_gather(pr, [eidx, c0])
                yr = plsc.load_gather(pr, [eidx, c1])
                zr = plsc.load_gather(pr, [eidx, c2])
                xc = plsc.load_gather(pc, [eidx, c0])
                yc = plsc.load_gather(pc, [eidx, c1])
                zc = plsc.load_gather(pc, [eidx, c2])
                bt = plsc.load_gather(bv, [eidx])
                dx = xr - xc
                dy = yr - yc
                dz = zr - zc
                s = dx * dx + dy * dy + dz * dz
                nrm = _sqrt16(s)
                e = plsc.load_gather(etab_v, [bt])
                r = nrm - e
                if not exact:
                    r = jnp.where((off + eidx) < e_total, r, jnp.float32(0.0))
                return acc + r * r

            return group

        pend_idx = [None] * (nchunks + 2)
        pend_g = [None] * (nchunks + 1)
        pend_idx[0] = start_idx(0)
        for cp in pend_idx[0]:
            cp.wait()
        pend_g[0] = start_gather(0)
        if nchunks > 1:
            pend_idx[1] = start_idx(1)

        acc = jnp.zeros((_L,), jnp.float32)
        for k in range(nchunks):
            if k + 1 < nchunks:
                for cp in pend_idx[k + 1]:
                    cp.wait()
                pend_g[k + 1] = start_gather(k + 1)
            for cp in pend_g[k]:
                cp.wait()
            if k + 2 < nchunks:
                pend_idx[k + 2] = start_idx(k + 2)
            acc = lax.fori_loop(0, groups, make_group(k % 2, base + k * chunk),
                                acc)

        acc_v[...] = acc
        pltpu.sync_copy(acc_v, out_hbm.at[wid])

    mesh = plsc.VectorSubcoreMesh(core_axis_name="c", subcore_axis_name="s",
                                  num_cores=_NC, num_subcores=_NS)
    return pl.kernel(
        body,
        out_type=jax.ShapeDtypeStruct((_NW, _L), jnp.float32),
        mesh=mesh,
        scratch_types=[
            pltpu.VMEM((chunk,), jnp.int32),
            pltpu.VMEM((chunk,), jnp.int32),
            pltpu.VMEM((chunk,), jnp.int32),
            pltpu.VMEM((chunk,), jnp.int32),
            pltpu.VMEM((chunk,), jnp.int32),
            pltpu.VMEM((chunk,), jnp.int32),
            pltpu.VMEM((chunk, 8), jnp.float32),
            pltpu.VMEM((chunk, 8), jnp.float32),
            pltpu.VMEM((chunk, 8), jnp.float32),
            pltpu.VMEM((chunk, 8), jnp.float32),
            pltpu.VMEM((_L,), jnp.float32),
            pltpu.VMEM((_L,), jnp.float32),
            pltpu.SemaphoreType.DMA,
            pltpu.SemaphoreType.DMA,
            pltpu.SemaphoreType.DMA,
            pltpu.SemaphoreType.DMA,
        ],
        compiler_params=pltpu.CompilerParams(needs_layout_passes=False,
                                             use_tc_tiling_on_sc=False),
    )


def kernel(positions, edge_index, bond_types, batch):
    del batch
    n_nodes = positions.shape[0]
    e_total = edge_index.shape[1]

    row = edge_index[0].astype(jnp.int32)
    col = edge_index[1].astype(jnp.int32)
    bt = bond_types.astype(jnp.int32)
    pos8 = jnp.concatenate(
        [positions.astype(jnp.float32), jnp.zeros((n_nodes, 5), jnp.float32)],
        axis=1)

    chunk = _pick_chunk(e_total // _NW) if e_total % _NW == 0 else 0
    if chunk:
        per_w = e_total // _NW
    else:
        chunk = 2048
        per_w = -(-e_total // (_NW * chunk)) * chunk
        e_pad = per_w * _NW
        row = jnp.pad(row, (0, e_pad - e_total))
        col = jnp.pad(col, (0, e_pad - e_total))
        bt = jnp.pad(bt, (0, e_pad - e_total))

    run = _build(n_nodes, e_total, chunk, per_w)
    partials = run(pos8, row, col, bt)
    return jnp.sum(partials) / jnp.float32(e_total)


# speedup vs baseline: 29.6585x; 1.1341x over previous
import jax
import jax.numpy as jnp
import numpy as np
from jax import lax
from jax.experimental import pallas as pl
from jax.experimental.pallas import tpu as pltpu
from jax.experimental.pallas import tpu_sc as plsc

_NC = 2
_NS = 16
_L = 16
_NW = _NC * _NS

_MAGIC = np.int32(0x5F3759DF)


def _sqrt16(s):
    i = lax.bitcast_convert_type(s, jnp.int32)
    i = _MAGIC - lax.shift_right_logical(i, 1)
    y = lax.bitcast_convert_type(i, jnp.float32)
    h = s * jnp.float32(0.5)
    for _ in range(3):
        y = y * (jnp.float32(1.5) - h * y * y)
    return s * y


def _pick_chunk(per_w):
    best = 0
    for c in range(16, 4097, 16):
        if per_w % c == 0:
            best = c
    return best


def _build(n_nodes, e_total, chunk, per_w):
    nchunks = per_w // chunk
    groups = chunk // _L
    exact = per_w * _NW == e_total

    def body(pos_hbm, edge_hbm, bt_hbm, out_hbm,
             idxr0, idxr1, idxc0, idxc1, btv0, btv1,
             posr0, posr1, posc0, posc1, acc_v, etab_v,
             sem_i0, sem_i1, sem_g0, sem_g1):
        idxr = (idxr0, idxr1)
        idxc = (idxc0, idxc1)
        btv = (btv0, btv1)
        posr = (posr0, posr1)
        posc = (posc0, posc1)
        sem_i = (sem_i0, sem_i1)
        sem_g = (sem_g0, sem_g1)

        wid = lax.axis_index("s") * _NC + lax.axis_index("c")
        base = wid * per_w
        iota = lax.iota(jnp.int32, _L)
        c0 = jnp.zeros((_L,), jnp.int32)
        c1 = jnp.full((_L,), 1, jnp.int32)
        c2 = jnp.full((_L,), 2, jnp.int32)
        etab_v[...] = jnp.where(iota == 1, jnp.float32(1.34),
                                jnp.where(iota == 2, jnp.float32(1.2),
                                          jnp.where(iota == 3, jnp.float32(1.4),
                                                    jnp.float32(1.54))))

        def start_idx(k):
            b = k % 2
            off = base + k * chunk
            return (
                pltpu.async_copy(edge_hbm.at[0, pl.ds(off, chunk)], idxr[b], sem_i[b]),
                pltpu.async_copy(edge_hbm.at[1, pl.ds(off, chunk)], idxc[b], sem_i[b]),
                pltpu.async_copy(bt_hbm.at[pl.ds(off, chunk)], btv[b], sem_i[b]),
            )

        def start_gather(k):
            b = k % 2
            return (
                pltpu.async_copy(pos_hbm.at[idxr[b]], posr[b], sem_g[b]),
                pltpu.async_copy(pos_hbm.at[idxc[b]], posc[b], sem_g[b]),
            )

        def make_group(b, off):
            pr, pc, bv = posr[b], posc[b], btv[b]

            def group(g, acc):
                eidx = g * _L + iota
                xr = plsc.load_gather(pr, [eidx, c0])
                yr = plsc.load_gather(pr, [eidx, c1])
                zr = plsc.load_gather(pr, [eidx, c2])
                xc = plsc.load_gather(pc, [eidx, c0])
                yc = plsc.load_gather(pc, [eidx, c1])
                zc = plsc.load_gather(pc, [eidx, c2])
                bt = plsc.load_gather(bv, [eidx])
                dx = xr - xc
                dy = yr - yc
                dz = zr - zc
                s = dx * dx + dy * dy + dz * dz
                nrm = _sqrt16(s)
                e = plsc.load_gather(etab_v, [bt])
                r = nrm - e
                if not exact:
                    r = jnp.where((off + eidx) < e_total, r, jnp.float32(0.0))
                return acc + r * r

            return group

        pend_idx = [None] * (nchunks + 2)
        pend_g = [None] * (nchunks + 1)
        pend_idx[0] = start_idx(0)
        for cp in pend_idx[0]:
            cp.wait()
        pend_g[0] = start_gather(0)
        if nchunks > 1:
            pend_idx[1] = start_idx(1)

        acc = jnp.zeros((_L,), jnp.float32)
        for k in range(nchunks):
            if k + 1 < nchunks:
                for cp in pend_idx[k + 1]:
                    cp.wait()
                pend_g[k + 1] = start_gather(k + 1)
            for cp in pend_g[k]:
                cp.wait()
            if k + 2 < nchunks:
                pend_idx[k + 2] = start_idx(k + 2)
            acc = lax.fori_loop(0, groups, make_group(k % 2, base + k * chunk),
                                acc)

        acc_v[...] = acc
        pltpu.sync_copy(acc_v, out_hbm.at[wid])

    mesh = plsc.VectorSubcoreMesh(core_axis_name="c", subcore_axis_name="s",
                                  num_cores=_NC, num_subcores=_NS)
    return pl.kernel(
        body,
        out_type=jax.ShapeDtypeStruct((_NW, _L), jnp.float32),
        mesh=mesh,
        scratch_types=[
            pltpu.VMEM((chunk,), jnp.int32),
            pltpu.VMEM((chunk,), jnp.int32),
            pltpu.VMEM((chunk,), jnp.int32),
            pltpu.VMEM((chunk,), jnp.int32),
            pltpu.VMEM((chunk,), jnp.int32),
            pltpu.VMEM((chunk,), jnp.int32),
            pltpu.VMEM((chunk, 8), jnp.float32),
            pltpu.VMEM((chunk, 8), jnp.float32),
            pltpu.VMEM((chunk, 8), jnp.float32),
            pltpu.VMEM((chunk, 8), jnp.float32),
            pltpu.VMEM((_L,), jnp.float32),
            pltpu.VMEM((_L,), jnp.float32),
            pltpu.SemaphoreType.DMA,
            pltpu.SemaphoreType.DMA,
            pltpu.SemaphoreType.DMA,
            pltpu.SemaphoreType.DMA,
        ],
        compiler_params=pltpu.CompilerParams(needs_layout_passes=False,
                                             use_tc_tiling_on_sc=False),
    )


def kernel(positions, edge_index, bond_types, batch):
    del batch
    n_nodes = positions.shape[0]
    e_total = edge_index.shape[1]

    edge = edge_index.astype(jnp.int32)
    bt = bond_types.astype(jnp.int32)
    pos8 = jnp.concatenate(
        [positions.astype(jnp.float32), jnp.zeros((n_nodes, 5), jnp.float32)],
        axis=1)

    chunk = _pick_chunk(e_total // _NW) if e_total % _NW == 0 else 0
    if chunk:
        per_w = e_total // _NW
    else:
        chunk = 2048
        per_w = -(-e_total // (_NW * chunk)) * chunk
        e_pad = per_w * _NW
        edge = jnp.pad(edge, ((0, 0), (0, e_pad - e_total)))
        bt = jnp.pad(bt, (0, e_pad - e_total))

    run = _build(n_nodes, e_total, chunk, per_w)
    partials = run(pos8, edge, bt)
    return jnp.sum(partials) / jnp.float32(e_total)


# speedup vs baseline: 43.6903x; 1.4731x over previous
import jax
import jax.numpy as jnp
import numpy as np
from jax import lax
from jax.experimental import pallas as pl
from jax.experimental.pallas import tpu as pltpu
from jax.experimental.pallas import tpu_sc as plsc

_NC = 2
_NS = 16
_L = 16
_NW = _NC * _NS

_MAGIC = np.int32(0x5F3759DF)


def _sqrt16(s):
    i = lax.bitcast_convert_type(s, jnp.int32)
    i = _MAGIC - lax.shift_right_logical(i, 1)
    y = lax.bitcast_convert_type(i, jnp.float32)
    h = s * jnp.float32(0.5)
    for _ in range(3):
        y = y * (jnp.float32(1.5) - h * y * y)
    return s * y


def _pick_chunk(per_w):
    best = 0
    for c in range(16, 4097, 16):
        if per_w % c == 0:
            best = c
    return best


def _build_pad(n_nodes, rpw):
    last = n_nodes - (_NW - 1) * rpw

    def body(post_hbm, pos8_hbm, xs_v, ys_v, zs_v, buf8_v, sem):
        wid = lax.axis_index("s") * _NC + lax.axis_index("c")
        start = wid * rpw
        iota = lax.iota(jnp.int32, _L)
        c0 = jnp.zeros((_L,), jnp.int32)
        c1 = jnp.full((_L,), 1, jnp.int32)
        c2 = jnp.full((_L,), 2, jnp.int32)

        def run_rows(nrows):
            cps = (
                pltpu.async_copy(post_hbm.at[0, pl.ds(start, nrows)],
                                 xs_v.at[pl.ds(0, nrows)], sem),
                pltpu.async_copy(post_hbm.at[1, pl.ds(start, nrows)],
                                 ys_v.at[pl.ds(0, nrows)], sem),
                pltpu.async_copy(post_hbm.at[2, pl.ds(start, nrows)],
                                 zs_v.at[pl.ds(0, nrows)], sem),
            )
            for cp in cps:
                cp.wait()

            def group(g, carry):
                ridx = g * _L + iota
                off = g * _L
                plsc.store_scatter(buf8_v, [ridx, c0], xs_v[pl.ds(off, _L)])
                plsc.store_scatter(buf8_v, [ridx, c1], ys_v[pl.ds(off, _L)])
                plsc.store_scatter(buf8_v, [ridx, c2], zs_v[pl.ds(off, _L)])
                return carry

            lax.fori_loop(0, nrows // _L, group, jnp.int32(0))
            pltpu.sync_copy(buf8_v.at[pl.ds(0, nrows)],
                            pos8_hbm.at[pl.ds(start, nrows)])

        @pl.when(wid < _NW - 1)
        def _():
            run_rows(rpw)

        @pl.when(wid == _NW - 1)
        def _():
            run_rows(last)

    mesh = plsc.VectorSubcoreMesh(core_axis_name="c", subcore_axis_name="s",
                                  num_cores=_NC, num_subcores=_NS)
    return pl.kernel(
        body,
        out_type=jax.ShapeDtypeStruct((n_nodes, 8), jnp.float32),
        mesh=mesh,
        scratch_types=[
            pltpu.VMEM((rpw,), jnp.float32),
            pltpu.VMEM((rpw,), jnp.float32),
            pltpu.VMEM((rpw,), jnp.float32),
            pltpu.VMEM((rpw, 8), jnp.float32),
            pltpu.SemaphoreType.DMA,
        ],
        compiler_params=pltpu.CompilerParams(needs_layout_passes=False,
                                             use_tc_tiling_on_sc=False),
    )


def _build(n_nodes, e_total, chunk, per_w):
    nchunks = per_w // chunk
    groups = chunk // _L
    exact = per_w * _NW == e_total

    def body(pos_hbm, edge_hbm, bt_hbm, out_hbm,
             idxr0, idxr1, idxc0, idxc1, btv0, btv1,
             posr0, posr1, posc0, posc1, acc_v, etab_v,
             sem_i0, sem_i1, sem_g0, sem_g1):
        idxr = (idxr0, idxr1)
        idxc = (idxc0, idxc1)
        btv = (btv0, btv1)
        posr = (posr0, posr1)
        posc = (posc0, posc1)
        sem_i = (sem_i0, sem_i1)
        sem_g = (sem_g0, sem_g1)

        wid = lax.axis_index("s") * _NC + lax.axis_index("c")
        base = wid * per_w
        iota = lax.iota(jnp.int32, _L)
        c0 = jnp.zeros((_L,), jnp.int32)
        c1 = jnp.full((_L,), 1, jnp.int32)
        c2 = jnp.full((_L,), 2, jnp.int32)
        etab_v[...] = jnp.where(iota == 1, jnp.float32(1.34),
                                jnp.where(iota == 2, jnp.float32(1.2),
                                          jnp.where(iota == 3, jnp.float32(1.4),
                                                    jnp.float32(1.54))))

        def start_idx(k):
            b = k % 2
            off = base + k * chunk
            return (
                pltpu.async_copy(edge_hbm.at[0, pl.ds(off, chunk)], idxr[b], sem_i[b]),
                pltpu.async_copy(edge_hbm.at[1, pl.ds(off, chunk)], idxc[b], sem_i[b]),
                pltpu.async_copy(bt_hbm.at[pl.ds(off, chunk)], btv[b], sem_i[b]),
            )

        def start_gather(k):
            b = k % 2
            return (
                pltpu.async_copy(pos_hbm.at[idxr[b]], posr[b], sem_g[b]),
                pltpu.async_copy(pos_hbm.at[idxc[b]], posc[b], sem_g[b]),
            )

        def make_group(b, off):
            pr, pc, bv = posr[b], posc[b], btv[b]

            def group(g, acc):
                eidx = g * _L + iota
                xr = plsc.load_gather(pr, [eidx, c0])
                yr = plsc.load_gather(pr, [eidx, c1])
                zr = plsc.load_gather(pr, [eidx, c2])
                xc = plsc.load_gather(pc, [eidx, c0])
                yc = plsc.load_gather(pc, [eidx, c1])
                zc = plsc.load_gather(pc, [eidx, c2])
                bt = plsc.load_gather(bv, [eidx])
                dx = xr - xc
                dy = yr - yc
                dz = zr - zc
                s = dx * dx + dy * dy + dz * dz
                nrm = _sqrt16(s)
                e = plsc.load_gather(etab_v, [bt])
                r = nrm - e
                if not exact:
                    r = jnp.where((off + eidx) < e_total, r, jnp.float32(0.0))
                return acc + r * r

            return group

        pend_idx = [None] * (nchunks + 2)
        pend_g = [None] * (nchunks + 1)
        pend_idx[0] = start_idx(0)
        for cp in pend_idx[0]:
            cp.wait()
        pend_g[0] = start_gather(0)
        if nchunks > 1:
            pend_idx[1] = start_idx(1)

        acc = jnp.zeros((_L,), jnp.float32)
        for k in range(nchunks):
            if k + 1 < nchunks:
                for cp in pend_idx[k + 1]:
                    cp.wait()
                pend_g[k + 1] = start_gather(k + 1)
            for cp in pend_g[k]:
                cp.wait()
            if k + 2 < nchunks:
                pend_idx[k + 2] = start_idx(k + 2)
            acc = lax.fori_loop(0, groups, make_group(k % 2, base + k * chunk),
                                acc)

        acc_v[...] = acc
        pltpu.sync_copy(acc_v, out_hbm.at[wid])

    mesh = plsc.VectorSubcoreMesh(core_axis_name="c", subcore_axis_name="s",
                                  num_cores=_NC, num_subcores=_NS)
    return pl.kernel(
        body,
        out_type=jax.ShapeDtypeStruct((_NW, _L), jnp.float32),
        mesh=mesh,
        scratch_types=[
            pltpu.VMEM((chunk,), jnp.int32),
            pltpu.VMEM((chunk,), jnp.int32),
            pltpu.VMEM((chunk,), jnp.int32),
            pltpu.VMEM((chunk,), jnp.int32),
            pltpu.VMEM((chunk,), jnp.int32),
            pltpu.VMEM((chunk,), jnp.int32),
            pltpu.VMEM((chunk, 8), jnp.float32),
            pltpu.VMEM((chunk, 8), jnp.float32),
            pltpu.VMEM((chunk, 8), jnp.float32),
            pltpu.VMEM((chunk, 8), jnp.float32),
            pltpu.VMEM((_L,), jnp.float32),
            pltpu.VMEM((_L,), jnp.float32),
            pltpu.SemaphoreType.DMA,
            pltpu.SemaphoreType.DMA,
            pltpu.SemaphoreType.DMA,
            pltpu.SemaphoreType.DMA,
        ],
        compiler_params=pltpu.CompilerParams(needs_layout_passes=False,
                                             use_tc_tiling_on_sc=False),
    )


def kernel(positions, edge_index, bond_types, batch):
    del batch
    n_nodes = positions.shape[0]
    e_total = edge_index.shape[1]

    edge = edge_index.astype(jnp.int32)
    bt = bond_types.astype(jnp.int32)

    rpw = 16 * (-(-n_nodes // (16 * _NW)))
    last = n_nodes - (_NW - 1) * rpw
    if 0 < last <= rpw and last % _L == 0:
        pos8 = _build_pad(n_nodes, rpw)(positions.T.astype(jnp.float32))
    else:
        pos8 = jnp.concatenate(
            [positions.astype(jnp.float32),
             jnp.zeros((n_nodes, 5), jnp.float32)], axis=1)

    chunk = _pick_chunk(e_total // _NW) if e_total % _NW == 0 else 0
    if chunk:
        per_w = e_total // _NW
    else:
        chunk = 2048
        per_w = -(-e_total // (_NW * chunk)) * chunk
        e_pad = per_w * _NW
        edge = jnp.pad(edge, ((0, 0), (0, e_pad - e_total)))
        bt = jnp.pad(bt, (0, e_pad - e_total))

    run = _build(n_nodes, e_total, chunk, per_w)
    partials = run(pos8, edge, bt)
    return jnp.sum(partials) / jnp.float32(e_total)


# speedup vs baseline: 45.9508x; 1.0517x over previous
import jax
import jax.numpy as jnp
import numpy as np
from jax import lax
from jax.experimental import pallas as pl
from jax.experimental.pallas import tpu as pltpu
from jax.experimental.pallas import tpu_sc as plsc

_NC = 2
_NS = 16
_L = 16
_NW = _NC * _NS

_MAGIC = np.int32(0x5F3759DF)


def _sqrt16(s):
    i = lax.bitcast_convert_type(s, jnp.int32)
    i = _MAGIC - lax.shift_right_logical(i, 1)
    y = lax.bitcast_convert_type(i, jnp.float32)
    h = s * jnp.float32(0.5)
    for _ in range(3):
        y = y * (jnp.float32(1.5) - h * y * y)
    return s * y


def _pick_chunk(per_w):
    best = 0
    for c in range(16, 4097, 16):
        if per_w % c == 0:
            best = c
    return best


def _build_pad(n_nodes, rpw):
    last = n_nodes - (_NW - 1) * rpw

    def body(post_hbm, pos8_hbm, xs_v, ys_v, zs_v, buf8_v, sem):
        wid = lax.axis_index("s") * _NC + lax.axis_index("c")
        start = wid * rpw
        iota = lax.iota(jnp.int32, _L)
        c0 = jnp.zeros((_L,), jnp.int32)
        c1 = jnp.full((_L,), 1, jnp.int32)
        c2 = jnp.full((_L,), 2, jnp.int32)

        def run_rows(nrows):
            cps = (
                pltpu.async_copy(post_hbm.at[0, pl.ds(start, nrows)],
                                 xs_v.at[pl.ds(0, nrows)], sem),
                pltpu.async_copy(post_hbm.at[1, pl.ds(start, nrows)],
                                 ys_v.at[pl.ds(0, nrows)], sem),
                pltpu.async_copy(post_hbm.at[2, pl.ds(start, nrows)],
                                 zs_v.at[pl.ds(0, nrows)], sem),
            )
            for cp in cps:
                cp.wait()

            def group(g, carry):
                ridx = g * _L + iota
                off = g * _L
                plsc.store_scatter(buf8_v, [ridx, c0], xs_v[pl.ds(off, _L)])
                plsc.store_scatter(buf8_v, [ridx, c1], ys_v[pl.ds(off, _L)])
                plsc.store_scatter(buf8_v, [ridx, c2], zs_v[pl.ds(off, _L)])
                return carry

            lax.fori_loop(0, nrows // _L, group, jnp.int32(0))
            pltpu.sync_copy(buf8_v.at[pl.ds(0, nrows)],
                            pos8_hbm.at[pl.ds(start, nrows)])

        @pl.when(wid < _NW - 1)
        def _():
            run_rows(rpw)

        @pl.when(wid == _NW - 1)
        def _():
            run_rows(last)

    mesh = plsc.VectorSubcoreMesh(core_axis_name="c", subcore_axis_name="s",
                                  num_cores=_NC, num_subcores=_NS)
    return pl.kernel(
        body,
        out_type=jax.ShapeDtypeStruct((n_nodes, 8), jnp.float32),
        mesh=mesh,
        scratch_types=[
            pltpu.VMEM((rpw,), jnp.float32),
            pltpu.VMEM((rpw,), jnp.float32),
            pltpu.VMEM((rpw,), jnp.float32),
            pltpu.VMEM((rpw, 8), jnp.float32),
            pltpu.SemaphoreType.DMA,
        ],
        compiler_params=pltpu.CompilerParams(needs_layout_passes=False,
                                             use_tc_tiling_on_sc=False),
    )


def _build_tiles(e_total):
    T = e_total // 128
    base, rem = divmod(T, _NW)

    def body(pos_hbm, e3_hbm, bt_hbm, out_hbm,
             ebuf0, ebuf1, idxr0, idxr1, idxc0, idxc1, btv0, btv1,
             posr0, posr1, posc0, posc1, acc_v, etab_v,
             sem_i0, sem_i1, sem_g0, sem_g1):
        ebuf = (ebuf0, ebuf1)
        idxr = (idxr0, idxr1)
        idxc = (idxc0, idxc1)
        btv = (btv0, btv1)
        posr = (posr0, posr1)
        posc = (posc0, posc1)
        sem_i = (sem_i0, sem_i1)
        sem_g = (sem_g0, sem_g1)

        wid = lax.axis_index("s") * _NC + lax.axis_index("c")
        iota = lax.iota(jnp.int32, _L)
        c0 = jnp.zeros((_L,), jnp.int32)
        c1 = jnp.full((_L,), 1, jnp.int32)
        c2 = jnp.full((_L,), 2, jnp.int32)
        etab_v[...] = jnp.where(iota == 1, jnp.float32(1.34),
                                jnp.where(iota == 2, jnp.float32(1.2),
                                          jnp.where(iota == 3, jnp.float32(1.4),
                                                    jnp.float32(1.54))))

        def make_group(b):
            pr, pc, bv = posr[b], posc[b], btv[b]

            def group(g, acc):
                eidx = g * _L + iota
                xr = plsc.load_gather(pr, [eidx, c0])
                yr = plsc.load_gather(pr, [eidx, c1])
                zr = plsc.load_gather(pr, [eidx, c2])
                xc = plsc.load_gather(pc, [eidx, c0])
                yc = plsc.load_gather(pc, [eidx, c1])
                zc = plsc.load_gather(pc, [eidx, c2])
                bt = plsc.load_gather(bv, [eidx])
                dx = xr - xc
                dy = yr - yc
                dz = zr - zc
                s = dx * dx + dy * dy + dz * dz
                nrm = _sqrt16(s)
                e = plsc.load_gather(etab_v, [bt])
                r = nrm - e
                return acc + r * r

            return group

        def run_range(t0, ntiles):
            nfull, tail = divmod(ntiles, 16)
            chunks = [(k * 16, 16) for k in range(nfull)]
            if tail:
                chunks.append((nfull * 16, tail))
            nch = len(chunks)

            def start_idx(k):
                b = k % 2
                toff, nt = chunks[k]
                t = t0 + toff
                de = ebuf[b] if nt == 16 else ebuf[b].at[pl.ds(0, nt)]
                return (
                    pltpu.async_copy(e3_hbm.at[pl.ds(t, nt)], de, sem_i[b]),
                    pltpu.async_copy(bt_hbm.at[pl.ds(t * 128, nt * 128)],
                                     btv[b].at[pl.ds(0, nt * 128)], sem_i[b]),
                )

            def deinterleave(k):
                b = k % 2
                _, nt = chunks[k]

                def tile_body(ti, carry):
                    for j in range(8):
                        o = pl.ds(j * _L, _L)
                        idxr[b][pl.ds(ti * 128 + j * _L, _L)] = ebuf[b][ti, 0, o]
                        idxc[b][pl.ds(ti * 128 + j * _L, _L)] = ebuf[b][ti, 1, o]
                    return carry

                lax.fori_loop(0, nt, tile_body, jnp.int32(0))

            def start_gather(k):
                b = k % 2
                _, nt = chunks[k]
                n = nt * 128
                si = idxr[b] if nt == 16 else idxr[b].at[pl.ds(0, n)]
                sc = idxc[b] if nt == 16 else idxc[b].at[pl.ds(0, n)]
                dr = posr[b] if nt == 16 else posr[b].at[pl.ds(0, n)]
                dc = posc[b] if nt == 16 else posc[b].at[pl.ds(0, n)]
                return (
                    pltpu.async_copy(pos_hbm.at[si], dr, sem_g[b]),
                    pltpu.async_copy(pos_hbm.at[sc], dc, sem_g[b]),
                )

            pend_idx = [None] * (nch + 2)
            pend_g = [None] * (nch + 1)
            pend_idx[0] = start_idx(0)
            for cp in pend_idx[0]:
                cp.wait()
            deinterleave(0)
            pend_g[0] = start_gather(0)
            if nch > 1:
                pend_idx[1] = start_idx(1)

            acc = jnp.zeros((_L,), jnp.float32)
            for k in range(nch):
                if k + 1 < nch:
                    for cp in pend_idx[k + 1]:
                        cp.wait()
                    deinterleave(k + 1)
                    pend_g[k + 1] = start_gather(k + 1)
                for cp in pend_g[k]:
                    cp.wait()
                if k + 2 < nch:
                    pend_idx[k + 2] = start_idx(k + 2)
                acc = lax.fori_loop(0, chunks[k][1] * 8, make_group(k % 2), acc)

            acc_v[...] = acc

        run_range(wid * base, base)

        if rem:
            @pl.when(wid < rem)
            def _():
                te = _NW * base + wid
                cps = (
                    pltpu.async_copy(e3_hbm.at[pl.ds(te, 1)],
                                     ebuf0.at[pl.ds(0, 1)], sem_i0),
                    pltpu.async_copy(bt_hbm.at[pl.ds(te * 128, 128)],
                                     btv0.at[pl.ds(0, 128)], sem_i0),
                )
                for cp in cps:
                    cp.wait()
                for j in range(8):
                    o = pl.ds(j * _L, _L)
                    idxr0[pl.ds(j * _L, _L)] = ebuf0[0, 0, o]
                    idxc0[pl.ds(j * _L, _L)] = ebuf0[0, 1, o]
                g1 = pltpu.async_copy(pos_hbm.at[idxr0.at[pl.ds(0, 128)]],
                                      posr0.at[pl.ds(0, 128)], sem_g0)
                g2 = pltpu.async_copy(pos_hbm.at[idxc0.at[pl.ds(0, 128)]],
                                      posc0.at[pl.ds(0, 128)], sem_g0)
                g1.wait()
                g2.wait()
                acc2 = lax.fori_loop(0, 8, make_group(0),
                                     jnp.zeros((_L,), jnp.float32))
                acc_v[...] = acc_v[...] + acc2

        pltpu.sync_copy(acc_v, out_hbm.at[wid])

    mesh = plsc.VectorSubcoreMesh(core_axis_name="c", subcore_axis_name="s",
                                  num_cores=_NC, num_subcores=_NS)
    return pl.kernel(
        body,
        out_type=jax.ShapeDtypeStruct((_NW, _L), jnp.float32),
        mesh=mesh,
        scratch_types=[
            pltpu.VMEM((16, 2, 128), jnp.int32),
            pltpu.VMEM((16, 2, 128), jnp.int32),
            pltpu.VMEM((2048,), jnp.int32),
            pltpu.VMEM((2048,), jnp.int32),
            pltpu.VMEM((2048,), jnp.int32),
            pltpu.VMEM((2048,), jnp.int32),
            pltpu.VMEM((2048,), jnp.int32),
            pltpu.VMEM((2048,), jnp.int32),
            pltpu.VMEM((2048, 8), jnp.float32),
            pltpu.VMEM((2048, 8), jnp.float32),
            pltpu.VMEM((2048, 8), jnp.float32),
            pltpu.VMEM((2048, 8), jnp.float32),
            pltpu.VMEM((_L,), jnp.float32),
            pltpu.VMEM((_L,), jnp.float32),
            pltpu.SemaphoreType.DMA,
            pltpu.SemaphoreType.DMA,
            pltpu.SemaphoreType.DMA,
            pltpu.SemaphoreType.DMA,
        ],
        compiler_params=pltpu.CompilerParams(needs_layout_passes=False,
                                             use_tc_tiling_on_sc=False),
    )


def _build(n_nodes, e_total, chunk, per_w):
    nchunks = per_w // chunk
    groups = chunk // _L
    exact = per_w * _NW == e_total

    def body(pos_hbm, edge_hbm, bt_hbm, out_hbm,
             idxr0, idxr1, idxc0, idxc1, btv0, btv1,
             posr0, posr1, posc0, posc1, acc_v, etab_v,
             sem_i0, sem_i1, sem_g0, sem_g1):
        idxr = (idxr0, idxr1)
        idxc = (idxc0, idxc1)
        btv = (btv0, btv1)
        posr = (posr0, posr1)
        posc = (posc0, posc1)
        sem_i = (sem_i0, sem_i1)
        sem_g = (sem_g0, sem_g1)

        wid = lax.axis_index("s") * _NC + lax.axis_index("c")
        base = wid * per_w
        iota = lax.iota(jnp.int32, _L)
        c0 = jnp.zeros((_L,), jnp.int32)
        c1 = jnp.full((_L,), 1, jnp.int32)
        c2 = jnp.full((_L,), 2, jnp.int32)
        etab_v[...] = jnp.where(iota == 1, jnp.float32(1.34),
                                jnp.where(iota == 2, jnp.float32(1.2),
                                          jnp.where(iota == 3, jnp.float32(1.4),
                                                    jnp.float32(1.54))))

        def start_idx(k):
            b = k % 2
            off = base + k * chunk
            return (
                pltpu.async_copy(edge_hbm.at[0, pl.ds(off, chunk)], idxr[b], sem_i[b]),
                pltpu.async_copy(edge_hbm.at[1, pl.ds(off, chunk)], idxc[b], sem_i[b]),
                pltpu.async_copy(bt_hbm.at[pl.ds(off, chunk)], btv[b], sem_i[b]),
            )

        def start_gather(k):
            b = k % 2
            return (
                pltpu.async_copy(pos_hbm.at[idxr[b]], posr[b], sem_g[b]),
                pltpu.async_copy(pos_hbm.at[idxc[b]], posc[b], sem_g[b]),
            )

        def make_group(b, off):
            pr, pc, bv = posr[b], posc[b], btv[b]

            def group(g, acc):
                eidx = g * _L + iota
                xr = plsc.load_gather(pr, [eidx, c0])
                yr = plsc.load_gather(pr, [eidx, c1])
                zr = plsc.load_gather(pr, [eidx, c2])
                xc = plsc.load_gather(pc, [eidx, c0])
                yc = plsc.load_gather(pc, [eidx, c1])
                zc = plsc.load_gather(pc, [eidx, c2])
                bt = plsc.load_gather(bv, [eidx])
                dx = xr - xc
                dy = yr - yc
                dz = zr - zc
                s = dx * dx + dy * dy + dz * dz
                nrm = _sqrt16(s)
                e = plsc.load_gather(etab_v, [bt])
                r = nrm - e
                if not exact:
                    r = jnp.where((off + eidx) < e_total, r, jnp.float32(0.0))
                return acc + r * r

            return group

        pend_idx = [None] * (nchunks + 2)
        pend_g = [None] * (nchunks + 1)
        pend_idx[0] = start_idx(0)
        for cp in pend_idx[0]:
            cp.wait()
        pend_g[0] = start_gather(0)
        if nchunks > 1:
            pend_idx[1] = start_idx(1)

        acc = jnp.zeros((_L,), jnp.float32)
        for k in range(nchunks):
            if k + 1 < nchunks:
                for cp in pend_idx[k + 1]:
                    cp.wait()
                pend_g[k + 1] = start_gather(k + 1)
            for cp in pend_g[k]:
                cp.wait()
            if k + 2 < nchunks:
                pend_idx[k + 2] = start_idx(k + 2)
            acc = lax.fori_loop(0, groups, make_group(k % 2, base + k * chunk),
                                acc)

        acc_v[...] = acc
        pltpu.sync_copy(acc_v, out_hbm.at[wid])

    mesh = plsc.VectorSubcoreMesh(core_axis_name="c", subcore_axis_name="s",
                                  num_cores=_NC, num_subcores=_NS)
    return pl.kernel(
        body,
        out_type=jax.ShapeDtypeStruct((_NW, _L), jnp.float32),
        mesh=mesh,
        scratch_types=[
            pltpu.VMEM((chunk,), jnp.int32),
            pltpu.VMEM((chunk,), jnp.int32),
            pltpu.VMEM((chunk,), jnp.int32),
            pltpu.VMEM((chunk,), jnp.int32),
            pltpu.VMEM((chunk,), jnp.int32),
            pltpu.VMEM((chunk,), jnp.int32),
            pltpu.VMEM((chunk, 8), jnp.float32),
            pltpu.VMEM((chunk, 8), jnp.float32),
            pltpu.VMEM((chunk, 8), jnp.float32),
            pltpu.VMEM((chunk, 8), jnp.float32),
            pltpu.VMEM((_L,), jnp.float32),
            pltpu.VMEM((_L,), jnp.float32),
            pltpu.SemaphoreType.DMA,
            pltpu.SemaphoreType.DMA,
            pltpu.SemaphoreType.DMA,
            pltpu.SemaphoreType.DMA,
        ],
        compiler_params=pltpu.CompilerParams(needs_layout_passes=False,
                                             use_tc_tiling_on_sc=False),
    )


def kernel(positions, edge_index, bond_types, batch):
    del batch
    n_nodes = positions.shape[0]
    e_total = edge_index.shape[1]

    edge = edge_index.astype(jnp.int32)
    bt = bond_types.astype(jnp.int32)

    rpw = 16 * (-(-n_nodes // (16 * _NW)))
    last = n_nodes - (_NW - 1) * rpw
    if 0 < last <= rpw and last % _L == 0:
        pos8 = _build_pad(n_nodes, rpw)(positions.T.astype(jnp.float32))
    else:
        pos8 = jnp.concatenate(
            [positions.astype(jnp.float32),
             jnp.zeros((n_nodes, 5), jnp.float32)], axis=1)

    if e_total % 128 == 0 and e_total // 128 >= _NW:
        e3 = jnp.transpose(edge.reshape(2, e_total // 128, 128), (1, 0, 2))
        partials = _build_tiles(e_total)(pos8, e3, bt)
    else:
        chunk = _pick_chunk(e_total // _NW) if e_total % _NW == 0 else 0
        if chunk:
            per_w = e_total // _NW
        else:
            chunk = 2048
            per_w = -(-e_total // (_NW * chunk)) * chunk
            e_pad = per_w * _NW
            edge = jnp.pad(edge, ((0, 0), (0, e_pad - e_total)))
            bt = jnp.pad(bt, (0, e_pad - e_total))
        run = _build(n_nodes, e_total, chunk, per_w)
        partials = run(pos8, edge, bt)
    return jnp.sum(partials) / jnp.float32(e_total)


# speedup vs baseline: 46.2244x; 1.0060x over previous
import jax
import jax.numpy as jnp
import numpy as np
from jax import lax
from jax.experimental import pallas as pl
from jax.experimental.pallas import tpu as pltpu
from jax.experimental.pallas import tpu_sc as plsc

_NC = 2
_NS = 16
_L = 16
_NW = _NC * _NS

_MAGIC = np.int32(0x5F3759DF)


def _sqrt16(s):
    i = lax.bitcast_convert_type(s, jnp.int32)
    i = _MAGIC - lax.shift_right_logical(i, 1)
    y = lax.bitcast_convert_type(i, jnp.float32)
    h = s * jnp.float32(0.5)
    for _ in range(2):
        y = y * (jnp.float32(1.5) - h * y * y)
    return s * y


def _pick_chunk(per_w):
    best = 0
    for c in range(16, 4097, 16):
        if per_w % c == 0:
            best = c
    return best


def _build_pad(n_nodes, rpw):
    last = n_nodes - (_NW - 1) * rpw

    def body(post_hbm, pos8_hbm, xs_v, ys_v, zs_v, buf8_v, sem):
        wid = lax.axis_index("s") * _NC + lax.axis_index("c")
        start = wid * rpw
        iota = lax.iota(jnp.int32, _L)
        c0 = jnp.zeros((_L,), jnp.int32)
        c1 = jnp.full((_L,), 1, jnp.int32)
        c2 = jnp.full((_L,), 2, jnp.int32)

        def run_rows(nrows):
            cps = (
                pltpu.async_copy(post_hbm.at[0, pl.ds(start, nrows)],
                                 xs_v.at[pl.ds(0, nrows)], sem),
                pltpu.async_copy(post_hbm.at[1, pl.ds(start, nrows)],
                                 ys_v.at[pl.ds(0, nrows)], sem),
                pltpu.async_copy(post_hbm.at[2, pl.ds(start, nrows)],
                                 zs_v.at[pl.ds(0, nrows)], sem),
            )
            for cp in cps:
                cp.wait()

            def group(g, carry):
                ridx = g * _L + iota
                off = g * _L
                plsc.store_scatter(buf8_v, [ridx, c0], xs_v[pl.ds(off, _L)])
                plsc.store_scatter(buf8_v, [ridx, c1], ys_v[pl.ds(off, _L)])
                plsc.store_scatter(buf8_v, [ridx, c2], zs_v[pl.ds(off, _L)])
                return carry

            lax.fori_loop(0, nrows // _L, group, jnp.int32(0))
            pltpu.sync_copy(buf8_v.at[pl.ds(0, nrows)],
                            pos8_hbm.at[pl.ds(start, nrows)])

        @pl.when(wid < _NW - 1)
        def _():
            run_rows(rpw)

        @pl.when(wid == _NW - 1)
        def _():
            run_rows(last)

    mesh = plsc.VectorSubcoreMesh(core_axis_name="c", subcore_axis_name="s",
                                  num_cores=_NC, num_subcores=_NS)
    return pl.kernel(
        body,
        out_type=jax.ShapeDtypeStruct((n_nodes, 8), jnp.float32),
        mesh=mesh,
        scratch_types=[
            pltpu.VMEM((rpw,), jnp.float32),
            pltpu.VMEM((rpw,), jnp.float32),
            pltpu.VMEM((rpw,), jnp.float32),
            pltpu.VMEM((rpw, 8), jnp.float32),
            pltpu.SemaphoreType.DMA,
        ],
        compiler_params=pltpu.CompilerParams(needs_layout_passes=False,
                                             use_tc_tiling_on_sc=False),
    )


def _build_tiles(e_total):
    T = e_total // 128
    base, rem = divmod(T, _NW)

    def body(pos_hbm, e3_hbm, bt_hbm, out_hbm,
             ebuf0, ebuf1, idxr0, idxr1, idxc0, idxc1, btv0, btv1,
             posr0, posr1, posc0, posc1, acc_v, etab_v,
             sem_i0, sem_i1, sem_g0, sem_g1):
        ebuf = (ebuf0, ebuf1)
        idxr = (idxr0, idxr1)
        idxc = (idxc0, idxc1)
        btv = (btv0, btv1)
        posr = (posr0, posr1)
        posc = (posc0, posc1)
        sem_i = (sem_i0, sem_i1)
        sem_g = (sem_g0, sem_g1)

        wid = lax.axis_index("s") * _NC + lax.axis_index("c")
        iota = lax.iota(jnp.int32, _L)
        c0 = jnp.zeros((_L,), jnp.int32)
        c1 = jnp.full((_L,), 1, jnp.int32)
        c2 = jnp.full((_L,), 2, jnp.int32)
        etab_v[...] = jnp.where(iota == 1, jnp.float32(1.34),
                                jnp.where(iota == 2, jnp.float32(1.2),
                                          jnp.where(iota == 3, jnp.float32(1.4),
                                                    jnp.float32(1.54))))

        def make_group(b):
            pr, pc, bv = posr[b], posc[b], btv[b]

            def group(g, acc):
                eidx = g * _L + iota
                xr = plsc.load_gather(pr, [eidx, c0])
                yr = plsc.load_gather(pr, [eidx, c1])
                zr = plsc.load_gather(pr, [eidx, c2])
                xc = plsc.load_gather(pc, [eidx, c0])
                yc = plsc.load_gather(pc, [eidx, c1])
                zc = plsc.load_gather(pc, [eidx, c2])
                bt = plsc.load_gather(bv, [eidx])
                dx = xr - xc
                dy = yr - yc
                dz = zr - zc
                s = dx * dx + dy * dy + dz * dz
                nrm = _sqrt16(s)
                e = plsc.load_gather(etab_v, [bt])
                r = nrm - e
                return acc + r * r

            return group

        def run_range(t0, ntiles):
            nfull, tail = divmod(ntiles, 16)
            chunks = [(k * 16, 16) for k in range(nfull)]
            if tail:
                chunks.append((nfull * 16, tail))
            nch = len(chunks)

            def start_idx(k):
                b = k % 2
                toff, nt = chunks[k]
                t = t0 + toff
                de = ebuf[b] if nt == 16 else ebuf[b].at[pl.ds(0, nt)]
                return (
                    pltpu.async_copy(e3_hbm.at[pl.ds(t, nt)], de, sem_i[b]),
                    pltpu.async_copy(bt_hbm.at[pl.ds(t * 128, nt * 128)],
                                     btv[b].at[pl.ds(0, nt * 128)], sem_i[b]),
                )

            def deinterleave(k):
                b = k % 2
                _, nt = chunks[k]

                def tile_body(ti, carry):
                    for j in range(8):
                        o = pl.ds(j * _L, _L)
                        idxr[b][pl.ds(ti * 128 + j * _L, _L)] = ebuf[b][ti, 0, o]
                        idxc[b][pl.ds(ti * 128 + j * _L, _L)] = ebuf[b][ti, 1, o]
                    return carry

                lax.fori_loop(0, nt, tile_body, jnp.int32(0))

            def start_gather(k):
                b = k % 2
                _, nt = chunks[k]
                n = nt * 128
                si = idxr[b] if nt == 16 else idxr[b].at[pl.ds(0, n)]
                sc = idxc[b] if nt == 16 else idxc[b].at[pl.ds(0, n)]
                dr = posr[b] if nt == 16 else posr[b].at[pl.ds(0, n)]
                dc = posc[b] if nt == 16 else posc[b].at[pl.ds(0, n)]
                return (
                    pltpu.async_copy(pos_hbm.at[si], dr, sem_g[b]),
                    pltpu.async_copy(pos_hbm.at[sc], dc, sem_g[b]),
                )

            pend_idx = [None] * (nch + 2)
            pend_g = [None] * (nch + 1)
            pend_idx[0] = start_idx(0)
            for cp in pend_idx[0]:
                cp.wait()
            deinterleave(0)
            pend_g[0] = start_gather(0)
            if nch > 1:
                pend_idx[1] = start_idx(1)

            acc = jnp.zeros((_L,), jnp.float32)
            for k in range(nch):
                if k + 1 < nch:
                    for cp in pend_idx[k + 1]:
                        cp.wait()
                    deinterleave(k + 1)
                    pend_g[k + 1] = start_gather(k + 1)
                for cp in pend_g[k]:
                    cp.wait()
                if k + 2 < nch:
                    pend_idx[k + 2] = start_idx(k + 2)
                acc = acc + lax.fori_loop(0, chunks[k][1] * 8,
                                          make_group(k % 2),
                                          jnp.zeros((_L,), jnp.float32))

            acc_v[...] = acc

        run_range(wid * base, base)

        if rem:
            @pl.when(wid < rem)
            def _():
                te = _NW * base + wid
                cps = (
                    pltpu.async_copy(e3_hbm.at[pl.ds(te, 1)],
                                     ebuf0.at[pl.ds(0, 1)], sem_i0),
                    pltpu.async_copy(bt_hbm.at[pl.ds(te * 128, 128)],
                                     btv0.at[pl.ds(0, 128)], sem_i0),
                )
                for cp in cps:
                    cp.wait()
                for j in range(8):
                    o = pl.ds(j * _L, _L)
                    idxr0[pl.ds(j * _L, _L)] = ebuf0[0, 0, o]
                    idxc0[pl.ds(j * _L, _L)] = ebuf0[0, 1, o]
                g1 = pltpu.async_copy(pos_hbm.at[idxr0.at[pl.ds(0, 128)]],
                                      posr0.at[pl.ds(0, 128)], sem_g0)
                g2 = pltpu.async_copy(pos_hbm.at[idxc0.at[pl.ds(0, 128)]],
                                      posc0.at[pl.ds(0, 128)], sem_g0)
                g1.wait()
                g2.wait()
                acc2 = lax.fori_loop(0, 8, make_group(0),
                                     jnp.zeros((_L,), jnp.float32))
                acc_v[...] = acc_v[...] + acc2

        pltpu.sync_copy(acc_v, out_hbm.at[wid])

    mesh = plsc.VectorSubcoreMesh(core_axis_name="c", subcore_axis_name="s",
                                  num_cores=_NC, num_subcores=_NS)
    return pl.kernel(
        body,
        out_type=jax.ShapeDtypeStruct((_NW, _L), jnp.float32),
        mesh=mesh,
        scratch_types=[
            pltpu.VMEM((16, 2, 128), jnp.int32),
            pltpu.VMEM((16, 2, 128), jnp.int32),
            pltpu.VMEM((2048,), jnp.int32),
            pltpu.VMEM((2048,), jnp.int32),
            pltpu.VMEM((2048,), jnp.int32),
            pltpu.VMEM((2048,), jnp.int32),
            pltpu.VMEM((2048,), jnp.int32),
            pltpu.VMEM((2048,), jnp.int32),
            pltpu.VMEM((2048, 8), jnp.float32),
            pltpu.VMEM((2048, 8), jnp.float32),
            pltpu.VMEM((2048, 8), jnp.float32),
            pltpu.VMEM((2048, 8), jnp.float32),
            pltpu.VMEM((_L,), jnp.float32),
            pltpu.VMEM((_L,), jnp.float32),
            pltpu.SemaphoreType.DMA,
            pltpu.SemaphoreType.DMA,
            pltpu.SemaphoreType.DMA,
            pltpu.SemaphoreType.DMA,
        ],
        compiler_params=pltpu.CompilerParams(needs_layout_passes=False,
                                             use_tc_tiling_on_sc=False),
    )


def _build(n_nodes, e_total, chunk, per_w):
    nchunks = per_w // chunk
    groups = chunk // _L
    exact = per_w * _NW == e_total

    def body(pos_hbm, edge_hbm, bt_hbm, out_hbm,
             idxr0, idxr1, idxc0, idxc1, btv0, btv1,
             posr0, posr1, posc0, posc1, acc_v, etab_v,
             sem_i0, sem_i1, sem_g0, sem_g1):
        idxr = (idxr0, idxr1)
        idxc = (idxc0, idxc1)
        btv = (btv0, btv1)
        posr = (posr0, posr1)
        posc = (posc0, posc1)
        sem_i = (sem_i0, sem_i1)
        sem_g = (sem_g0, sem_g1)

        wid = lax.axis_index("s") * _NC + lax.axis_index("c")
        base = wid * per_w
        iota = lax.iota(jnp.int32, _L)
        c0 = jnp.zeros((_L,), jnp.int32)
        c1 = jnp.full((_L,), 1, jnp.int32)
        c2 = jnp.full((_L,), 2, jnp.int32)
        etab_v[...] = jnp.where(iota == 1, jnp.float32(1.34),
                                jnp.where(iota == 2, jnp.float32(1.2),
                                          jnp.where(iota == 3, jnp.float32(1.4),
                                                    jnp.float32(1.54))))

        def start_idx(k):
            b = k % 2
            off = base + k * chunk
            return (
                pltpu.async_copy(edge_hbm.at[0, pl.ds(off, chunk)], idxr[b], sem_i[b]),
                pltpu.async_copy(edge_hbm.at[1, pl.ds(off, chunk)], idxc[b], sem_i[b]),
                pltpu.async_copy(bt_hbm.at[pl.ds(off, chunk)], btv[b], sem_i[b]),
            )

        def start_gather(k):
            b = k % 2
            return (
                pltpu.async_copy(pos_hbm.at[idxr[b]], posr[b], sem_g[b]),
                pltpu.async_copy(pos_hbm.at[idxc[b]], posc[b], sem_g[b]),
            )

        def make_group(b, off):
            pr, pc, bv = posr[b], posc[b], btv[b]

            def group(g, acc):
                eidx = g * _L + iota
                xr = plsc.load_gather(pr, [eidx, c0])
                yr = plsc.load_gather(pr, [eidx, c1])
                zr = plsc.load_gather(pr, [eidx, c2])
                xc = plsc.load_gather(pc, [eidx, c0])
                yc = plsc.load_gather(pc, [eidx, c1])
                zc = plsc.load_gather(pc, [eidx, c2])
                bt = plsc.load_gather(bv, [eidx])
                dx = xr - xc
                dy = yr - yc
                dz = zr - zc
                s = dx * dx + dy * dy + dz * dz
                nrm = _sqrt16(s)
                e = plsc.load_gather(etab_v, [bt])
                r = nrm - e
                if not exact:
                    r = jnp.where((off + eidx) < e_total, r, jnp.float32(0.0))
                return acc + r * r

            return group

        pend_idx = [None] * (nchunks + 2)
        pend_g = [None] * (nchunks + 1)
        pend_idx[0] = start_idx(0)
        for cp in pend_idx[0]:
            cp.wait()
        pend_g[0] = start_gather(0)
        if nchunks > 1:
            pend_idx[1] = start_idx(1)

        acc = jnp.zeros((_L,), jnp.float32)
        for k in range(nchunks):
            if k + 1 < nchunks:
                for cp in pend_idx[k + 1]:
                    cp.wait()
                pend_g[k + 1] = start_gather(k + 1)
            for cp in pend_g[k]:
                cp.wait()
            if k + 2 < nchunks:
                pend_idx[k + 2] = start_idx(k + 2)
            acc = lax.fori_loop(0, groups, make_group(k % 2, base + k * chunk),
                                acc)

        acc_v[...] = acc
        pltpu.sync_copy(acc_v, out_hbm.at[wid])

    mesh = plsc.VectorSubcoreMesh(core_axis_name="c", subcore_axis_name="s",
                                  num_cores=_NC, num_subcores=_NS)
    return pl.kernel(
        body,
        out_type=jax.ShapeDtypeStruct((_NW, _L), jnp.float32),
        mesh=mesh,
        scratch_types=[
            pltpu.VMEM((chunk,), jnp.int32),
            pltpu.VMEM((chunk,), jnp.int32),
            pltpu.VMEM((chunk,), jnp.int32),
            pltpu.VMEM((chunk,), jnp.int32),
            pltpu.VMEM((chunk,), jnp.int32),
            pltpu.VMEM((chunk,), jnp.int32),
            pltpu.VMEM((chunk, 8), jnp.float32),
            pltpu.VMEM((chunk, 8), jnp.float32),
            pltpu.VMEM((chunk, 8), jnp.float32),
            pltpu.VMEM((chunk, 8), jnp.float32),
            pltpu.VMEM((_L,), jnp.float32),
            pltpu.VMEM((_L,), jnp.float32),
            pltpu.SemaphoreType.DMA,
            pltpu.SemaphoreType.DMA,
            pltpu.SemaphoreType.DMA,
            pltpu.SemaphoreType.DMA,
        ],
        compiler_params=pltpu.CompilerParams(needs_layout_passes=False,
                                             use_tc_tiling_on_sc=False),
    )


def kernel(positions, edge_index, bond_types, batch):
    del batch
    n_nodes = positions.shape[0]
    e_total = edge_index.shape[1]

    edge = edge_index.astype(jnp.int32)
    bt = bond_types.astype(jnp.int32)

    rpw = 16 * (-(-n_nodes // (16 * _NW)))
    last = n_nodes - (_NW - 1) * rpw
    if 0 < last <= rpw and last % _L == 0:
        pos8 = _build_pad(n_nodes, rpw)(positions.T.astype(jnp.float32))
    else:
        pos8 = jnp.concatenate(
            [positions.astype(jnp.float32),
             jnp.zeros((n_nodes, 5), jnp.float32)], axis=1)

    if e_total % 128 == 0 and e_total // 128 >= _NW:
        e3 = jnp.transpose(edge.reshape(2, e_total // 128, 128), (1, 0, 2))
        partials = _build_tiles(e_total)(pos8, e3, bt)
    else:
        chunk = _pick_chunk(e_total // _NW) if e_total % _NW == 0 else 0
        if chunk:
            per_w = e_total // _NW
        else:
            chunk = 2048
            per_w = -(-e_total // (_NW * chunk)) * chunk
            e_pad = per_w * _NW
            edge = jnp.pad(edge, ((0, 0), (0, e_pad - e_total)))
            bt = jnp.pad(bt, (0, e_pad - e_total))
        run = _build(n_nodes, e_total, chunk, per_w)
        partials = run(pos8, edge, bt)
    return jnp.sum(partials) / jnp.float32(e_total)


# speedup vs baseline: 81.0572x; 1.7536x over previous
import jax
import jax.numpy as jnp
import numpy as np
from jax import lax
from jax.experimental import pallas as pl
from jax.experimental.pallas import tpu as pltpu
from jax.experimental.pallas import tpu_sc as plsc

_NC = 2
_NS = 16
_L = 16
_NW = _NC * _NS

_MAGIC = np.int32(0x5F3759DF)


def _sqrt16(s):
    i = lax.bitcast_convert_type(s, jnp.int32)
    i = _MAGIC - lax.shift_right_logical(i, 1)
    y = lax.bitcast_convert_type(i, jnp.float32)
    h = s * jnp.float32(0.5)
    for _ in range(2):
        y = y * (jnp.float32(1.5) - h * y * y)
    return s * y


def _pick_chunk(per_w):
    best = 0
    for c in range(16, 4097, 16):
        if per_w % c == 0:
            best = c
    return best


def _build_pad(n_nodes, rpw):
    last = n_nodes - (_NW - 1) * rpw

    def body(post_hbm, pos8_hbm, xs_v, ys_v, zs_v, buf8_v, sem):
        wid = lax.axis_index("s") * _NC + lax.axis_index("c")
        start = wid * rpw
        iota = lax.iota(jnp.int32, _L)
        c0 = jnp.zeros((_L,), jnp.int32)
        c1 = jnp.full((_L,), 1, jnp.int32)
        c2 = jnp.full((_L,), 2, jnp.int32)

        def run_rows(nrows):
            cps = (
                pltpu.async_copy(post_hbm.at[0, pl.ds(start, nrows)],
                                 xs_v.at[pl.ds(0, nrows)], sem),
                pltpu.async_copy(post_hbm.at[1, pl.ds(start, nrows)],
                                 ys_v.at[pl.ds(0, nrows)], sem),
                pltpu.async_copy(post_hbm.at[2, pl.ds(start, nrows)],
                                 zs_v.at[pl.ds(0, nrows)], sem),
            )
            for cp in cps:
                cp.wait()

            def group(g, carry):
                ridx = g * _L + iota
                off = g * _L
                plsc.store_scatter(buf8_v, [ridx, c0], xs_v[pl.ds(off, _L)])
                plsc.store_scatter(buf8_v, [ridx, c1], ys_v[pl.ds(off, _L)])
                plsc.store_scatter(buf8_v, [ridx, c2], zs_v[pl.ds(off, _L)])
                return carry

            lax.fori_loop(0, nrows // _L, group, jnp.int32(0))
            pltpu.sync_copy(buf8_v.at[pl.ds(0, nrows)],
                            pos8_hbm.at[pl.ds(start, nrows)])

        @pl.when(wid < _NW - 1)
        def _():
            run_rows(rpw)

        @pl.when(wid == _NW - 1)
        def _():
            run_rows(last)

    mesh = plsc.VectorSubcoreMesh(core_axis_name="c", subcore_axis_name="s",
                                  num_cores=_NC, num_subcores=_NS)
    return pl.kernel(
        body,
        out_type=jax.ShapeDtypeStruct((n_nodes, 8), jnp.float32),
        mesh=mesh,
        scratch_types=[
            pltpu.VMEM((rpw,), jnp.float32),
            pltpu.VMEM((rpw,), jnp.float32),
            pltpu.VMEM((rpw,), jnp.float32),
            pltpu.VMEM((rpw, 8), jnp.float32),
            pltpu.SemaphoreType.DMA,
        ],
        compiler_params=pltpu.CompilerParams(needs_layout_passes=False,
                                             use_tc_tiling_on_sc=False),
    )


def _build_tiles(e_total, n_nodes):
    T = e_total // 128
    base, rem = divmod(T, _NW)
    CT = 14
    CE = CT * 128
    r_reg = 16 * (-(-n_nodes // (_NS * 16)))
    r_last = n_nodes - (_NS - 1) * r_reg

    def body(post_hbm, e3_hbm, bt_hbm, out_hbm,
             pos_sh,
             ebuf0, ebuf1, idxr0, idxr1, idxc0, idxc1, btv0, btv1,
             posr0, posr1, posc0, posc1, acc_v, etab_v,
             sem_i0, sem_i1, sem_g0, sem_g1):
        ebuf = (ebuf0, ebuf1)
        idxr = (idxr0, idxr1)
        idxc = (idxc0, idxc1)
        btv = (btv0, btv1)
        posr = (posr0, posr1)
        posc = (posc0, posc1)
        sem_i = (sem_i0, sem_i1)
        sem_g = (sem_g0, sem_g1)

        wid = lax.axis_index("s") * _NC + lax.axis_index("c")
        sid = lax.axis_index("s")
        iota = lax.iota(jnp.int32, _L)
        c0 = jnp.zeros((_L,), jnp.int32)
        c1 = jnp.full((_L,), 1, jnp.int32)
        c2 = jnp.full((_L,), 2, jnp.int32)
        def fill_range(tstart, nrows):
            nf, tl = divmod(nrows, CE)
            passes = [(q * CE, CE) for q in range(nf)]
            if tl:
                passes.append((nf * CE, tl))
            for rr, nn in passes:
                r0 = tstart + rr
                cps = tuple(
                    pltpu.async_copy(post_hbm.at[c, pl.ds(r0, nn)],
                                     pv.at[pl.ds(0, nn)], sem_i0)
                    for c, pv in ((0, idxr0), (1, idxc0), (2, btv0)))
                for cp in cps:
                    cp.wait()

                def fill(g, carry):
                    ridx = g * _L + iota
                    o = pl.ds(g * _L, _L)
                    plsc.store_scatter(
                        posr0, [ridx, c0],
                        plsc.bitcast(idxr0[o], jnp.float32))
                    plsc.store_scatter(
                        posr0, [ridx, c1],
                        plsc.bitcast(idxc0[o], jnp.float32))
                    plsc.store_scatter(
                        posr0, [ridx, c2],
                        plsc.bitcast(btv0[o], jnp.float32))
                    return carry

                lax.fori_loop(0, nn // _L, fill, jnp.int32(0))
                pltpu.sync_copy(posr0.at[pl.ds(0, nn)],
                                pos_sh.at[pl.ds(r0, nn)])

        @pl.when(sid < _NS - 1)
        def _():
            fill_range(sid * r_reg, r_reg)

        @pl.when(sid == _NS - 1)
        def _():
            fill_range((_NS - 1) * r_reg, r_last)

        plsc.subcore_barrier()
        etab_v[...] = jnp.where(iota == 1, jnp.float32(1.34),
                                jnp.where(iota == 2, jnp.float32(1.2),
                                          jnp.where(iota == 3, jnp.float32(1.4),
                                                    jnp.float32(1.54))))

        def make_group(b):
            pr, pc, bv = posr[b], posc[b], btv[b]

            def group(g, acc):
                eidx = g * _L + iota
                xr = plsc.load_gather(pr, [eidx, c0])
                yr = plsc.load_gather(pr, [eidx, c1])
                zr = plsc.load_gather(pr, [eidx, c2])
                xc = plsc.load_gather(pc, [eidx, c0])
                yc = plsc.load_gather(pc, [eidx, c1])
                zc = plsc.load_gather(pc, [eidx, c2])
                bt = plsc.load_gather(bv, [eidx])
                dx = xr - xc
                dy = yr - yc
                dz = zr - zc
                s = dx * dx + dy * dy + dz * dz
                nrm = _sqrt16(s)
                e = plsc.load_gather(etab_v, [bt])
                r = nrm - e
                return acc + r * r

            return group

        def run_range(t0, ntiles):
            nfull, tail = divmod(ntiles, CT)
            chunks = [(k * CT, CT) for k in range(nfull)]
            if tail:
                chunks.append((nfull * CT, tail))
            nch = len(chunks)

            def start_idx(k):
                b = k % 2
                toff, nt = chunks[k]
                t = t0 + toff
                de = ebuf[b] if nt == CT else ebuf[b].at[pl.ds(0, nt)]
                return (
                    pltpu.async_copy(e3_hbm.at[pl.ds(t, nt)], de, sem_i[b]),
                    pltpu.async_copy(bt_hbm.at[pl.ds(t * 128, nt * 128)],
                                     btv[b].at[pl.ds(0, nt * 128)], sem_i[b]),
                )

            def deinterleave(k):
                b = k % 2
                _, nt = chunks[k]

                def tile_body(ti, carry):
                    for j in range(8):
                        o = pl.ds(j * _L, _L)
                        idxr[b][pl.ds(ti * 128 + j * _L, _L)] = ebuf[b][ti, 0, o]
                        idxc[b][pl.ds(ti * 128 + j * _L, _L)] = ebuf[b][ti, 1, o]
                    return carry

                lax.fori_loop(0, nt, tile_body, jnp.int32(0))

            def start_gather(k):
                b = k % 2
                _, nt = chunks[k]
                n = nt * 128
                si = idxr[b] if nt == CT else idxr[b].at[pl.ds(0, n)]
                sc = idxc[b] if nt == CT else idxc[b].at[pl.ds(0, n)]
                dr = posr[b] if nt == CT else posr[b].at[pl.ds(0, n)]
                dc = posc[b] if nt == CT else posc[b].at[pl.ds(0, n)]
                return (
                    pltpu.async_copy(pos_sh.at[si], dr, sem_g[b]),
                    pltpu.async_copy(pos_sh.at[sc], dc, sem_g[b]),
                )

            pend_idx = [None] * (nch + 2)
            pend_g = [None] * (nch + 1)
            pend_idx[0] = start_idx(0)
            for cp in pend_idx[0]:
                cp.wait()
            deinterleave(0)
            pend_g[0] = start_gather(0)
            if nch > 1:
                pend_idx[1] = start_idx(1)

            acc = jnp.zeros((_L,), jnp.float32)
            for k in range(nch):
                if k + 1 < nch:
                    for cp in pend_idx[k + 1]:
                        cp.wait()
                    deinterleave(k + 1)
                    pend_g[k + 1] = start_gather(k + 1)
                for cp in pend_g[k]:
                    cp.wait()
                if k + 2 < nch:
                    pend_idx[k + 2] = start_idx(k + 2)
                acc = acc + lax.fori_loop(0, chunks[k][1] * 8,
                                          make_group(k % 2),
                                          jnp.zeros((_L,), jnp.float32))

            acc_v[...] = acc

        run_range(wid * base, base)

        if rem:
            @pl.when(wid < rem)
            def _():
                te = _NW * base + wid
                cps = (
                    pltpu.async_copy(e3_hbm.at[pl.ds(te, 1)],
                                     ebuf0.at[pl.ds(0, 1)], sem_i0),
                    pltpu.async_copy(bt_hbm.at[pl.ds(te * 128, 128)],
                                     btv0.at[pl.ds(0, 128)], sem_i0),
                )
                for cp in cps:
                    cp.wait()
                for j in range(8):
                    o = pl.ds(j * _L, _L)
                    idxr0[pl.ds(j * _L, _L)] = ebuf0[0, 0, o]
                    idxc0[pl.ds(j * _L, _L)] = ebuf0[0, 1, o]
                g1 = pltpu.async_copy(pos_sh.at[idxr0.at[pl.ds(0, 128)]],
                                      posr0.at[pl.ds(0, 128)], sem_g0)
                g2 = pltpu.async_copy(pos_sh.at[idxc0.at[pl.ds(0, 128)]],
                                      posc0.at[pl.ds(0, 128)], sem_g0)
                g1.wait()
                g2.wait()
                acc2 = lax.fori_loop(0, 8, make_group(0),
                                     jnp.zeros((_L,), jnp.float32))
                acc_v[...] = acc_v[...] + acc2

        pltpu.sync_copy(acc_v, out_hbm.at[wid])

    mesh = plsc.VectorSubcoreMesh(core_axis_name="c", subcore_axis_name="s",
                                  num_cores=_NC, num_subcores=_NS)
    return pl.kernel(
        body,
        out_type=jax.ShapeDtypeStruct((_NW, _L), jnp.float32),
        mesh=mesh,
        scratch_types=[
            pltpu.VMEM_SHARED((n_nodes, 8), jnp.float32),
            pltpu.VMEM((CT, 2, 128), jnp.int32),
            pltpu.VMEM((CT, 2, 128), jnp.int32),
            pltpu.VMEM((CE,), jnp.int32),
            pltpu.VMEM((CE,), jnp.int32),
            pltpu.VMEM((CE,), jnp.int32),
            pltpu.VMEM((CE,), jnp.int32),
            pltpu.VMEM((CE,), jnp.int32),
            pltpu.VMEM((CE,), jnp.int32),
            pltpu.VMEM((CE, 8), jnp.float32),
            pltpu.VMEM((CE, 8), jnp.float32),
            pltpu.VMEM((CE, 8), jnp.float32),
            pltpu.VMEM((CE, 8), jnp.float32),
            pltpu.VMEM((_L,), jnp.float32),
            pltpu.VMEM((_L,), jnp.float32),
            pltpu.SemaphoreType.DMA,
            pltpu.SemaphoreType.DMA,
            pltpu.SemaphoreType.DMA,
            pltpu.SemaphoreType.DMA,
        ],
        compiler_params=pltpu.CompilerParams(needs_layout_passes=False,
                                             use_tc_tiling_on_sc=False),
    )


def _build(n_nodes, e_total, chunk, per_w):
    nchunks = per_w // chunk
    groups = chunk // _L
    exact = per_w * _NW == e_total

    def body(pos_hbm, edge_hbm, bt_hbm, out_hbm,
             idxr0, idxr1, idxc0, idxc1, btv0, btv1,
             posr0, posr1, posc0, posc1, acc_v, etab_v,
             sem_i0, sem_i1, sem_g0, sem_g1):
        idxr = (idxr0, idxr1)
        idxc = (idxc0, idxc1)
        btv = (btv0, btv1)
        posr = (posr0, posr1)
        posc = (posc0, posc1)
        sem_i = (sem_i0, sem_i1)
        sem_g = (sem_g0, sem_g1)

        wid = lax.axis_index("s") * _NC + lax.axis_index("c")
        base = wid * per_w
        iota = lax.iota(jnp.int32, _L)
        c0 = jnp.zeros((_L,), jnp.int32)
        c1 = jnp.full((_L,), 1, jnp.int32)
        c2 = jnp.full((_L,), 2, jnp.int32)
        etab_v[...] = jnp.where(iota == 1, jnp.float32(1.34),
                                jnp.where(iota == 2, jnp.float32(1.2),
                                          jnp.where(iota == 3, jnp.float32(1.4),
                                                    jnp.float32(1.54))))

        def start_idx(k):
            b = k % 2
            off = base + k * chunk
            return (
                pltpu.async_copy(edge_hbm.at[0, pl.ds(off, chunk)], idxr[b], sem_i[b]),
                pltpu.async_copy(edge_hbm.at[1, pl.ds(off, chunk)], idxc[b], sem_i[b]),
                pltpu.async_copy(bt_hbm.at[pl.ds(off, chunk)], btv[b], sem_i[b]),
            )

        def start_gather(k):
            b = k % 2
            return (
                pltpu.async_copy(pos_sh.at[idxr[b]], posr[b], sem_g[b]),
                pltpu.async_copy(pos_sh.at[idxc[b]], posc[b], sem_g[b]),
            )

        def make_group(b, off):
            pr, pc, bv = posr[b], posc[b], btv[b]

            def group(g, acc):
                eidx = g * _L + iota
                xr = plsc.load_gather(pr, [eidx, c0])
                yr = plsc.load_gather(pr, [eidx, c1])
                zr = plsc.load_gather(pr, [eidx, c2])
                xc = plsc.load_gather(pc, [eidx, c0])
                yc = plsc.load_gather(pc, [eidx, c1])
                zc = plsc.load_gather(pc, [eidx, c2])
                bt = plsc.load_gather(bv, [eidx])
                dx = xr - xc
                dy = yr - yc
                dz = zr - zc
                s = dx * dx + dy * dy + dz * dz
                nrm = _sqrt16(s)
                e = plsc.load_gather(etab_v, [bt])
                r = nrm - e
                if not exact:
                    r = jnp.where((off + eidx) < e_total, r, jnp.float32(0.0))
                return acc + r * r

            return group

        pend_idx = [None] * (nchunks + 2)
        pend_g = [None] * (nchunks + 1)
        pend_idx[0] = start_idx(0)
        for cp in pend_idx[0]:
            cp.wait()
        pend_g[0] = start_gather(0)
        if nchunks > 1:
            pend_idx[1] = start_idx(1)

        acc = jnp.zeros((_L,), jnp.float32)
        for k in range(nchunks):
            if k + 1 < nchunks:
                for cp in pend_idx[k + 1]:
                    cp.wait()
                pend_g[k + 1] = start_gather(k + 1)
            for cp in pend_g[k]:
                cp.wait()
            if k + 2 < nchunks:
                pend_idx[k + 2] = start_idx(k + 2)
            acc = lax.fori_loop(0, groups, make_group(k % 2, base + k * chunk),
                                acc)

        acc_v[...] = acc
        pltpu.sync_copy(acc_v, out_hbm.at[wid])

    mesh = plsc.VectorSubcoreMesh(core_axis_name="c", subcore_axis_name="s",
                                  num_cores=_NC, num_subcores=_NS)
    return pl.kernel(
        body,
        out_type=jax.ShapeDtypeStruct((_NW, _L), jnp.float32),
        mesh=mesh,
        scratch_types=[
            pltpu.VMEM((chunk,), jnp.int32),
            pltpu.VMEM((chunk,), jnp.int32),
            pltpu.VMEM((chunk,), jnp.int32),
            pltpu.VMEM((chunk,), jnp.int32),
            pltpu.VMEM((chunk,), jnp.int32),
            pltpu.VMEM((chunk,), jnp.int32),
            pltpu.VMEM((chunk, 8), jnp.float32),
            pltpu.VMEM((chunk, 8), jnp.float32),
            pltpu.VMEM((chunk, 8), jnp.float32),
            pltpu.VMEM((chunk, 8), jnp.float32),
            pltpu.VMEM((_L,), jnp.float32),
            pltpu.VMEM((_L,), jnp.float32),
            pltpu.SemaphoreType.DMA,
            pltpu.SemaphoreType.DMA,
            pltpu.SemaphoreType.DMA,
            pltpu.SemaphoreType.DMA,
        ],
        compiler_params=pltpu.CompilerParams(needs_layout_passes=False,
                                             use_tc_tiling_on_sc=False),
    )


def kernel(positions, edge_index, bond_types, batch):
    del batch
    n_nodes = positions.shape[0]
    e_total = edge_index.shape[1]

    edge = edge_index.astype(jnp.int32)
    bt = bond_types.astype(jnp.int32)

    rpw = 16 * (-(-n_nodes // (16 * _NW)))
    last = n_nodes - (_NW - 1) * rpw
    if 0 < last <= rpw and last % _L == 0:
        pos8 = _build_pad(n_nodes, rpw)(positions.T.astype(jnp.float32))
    else:
        pos8 = jnp.concatenate(
            [positions.astype(jnp.float32),
             jnp.zeros((n_nodes, 5), jnp.float32)], axis=1)

    r_reg_w = 16 * (-(-n_nodes // (_NS * 16)))
    r_last_w = n_nodes - (_NS - 1) * r_reg_w
    if (e_total % 128 == 0 and e_total // 128 >= _NW
            and n_nodes % 16 == 0 and 0 < r_last_w <= r_reg_w):
        e3 = jnp.transpose(edge.reshape(2, e_total // 128, 128), (1, 0, 2))
        posTi = jnp.transpose(lax.bitcast_convert_type(
            positions.astype(jnp.float32), jnp.int32))
        partials = _build_tiles(e_total, n_nodes)(posTi, e3, bt)
    else:
        chunk = _pick_chunk(e_total // _NW) if e_total % _NW == 0 else 0
        if chunk:
            per_w = e_total // _NW
        else:
            chunk = 2048
            per_w = -(-e_total // (_NW * chunk)) * chunk
            e_pad = per_w * _NW
            edge = jnp.pad(edge, ((0, 0), (0, e_pad - e_total)))
            bt = jnp.pad(bt, (0, e_pad - e_total))
        run = _build(n_nodes, e_total, chunk, per_w)
        partials = run(pos8, edge, bt)
    return jnp.sum(partials) / jnp.float32(e_total)
